# Initial kernel scaffold; baseline (speedup 1.0000x reference)
import jax
import jax.numpy as jnp
from jax.experimental import pallas as pl


def kernel(image, edge_weights, angles, edge_features_1d, W_fe, b_fe, Wself1, Wmsg1, We1, be1, Wself2, Wmsg2, We2, be2, p_params, q_params, sp_indices, edge_index, round_n):
    raise NotImplementedError("write your pallas kernel here")


# speedup vs baseline: 3.7960x; 3.7960x over previous
import functools

import jax
import jax.numpy as jnp
from jax import lax
from jax.experimental import pallas as pl
from jax.experimental.pallas import tpu as pltpu
from jax.experimental.pallas import tpu_sc as plsc

F32 = jnp.float32
I32 = jnp.int32

N_NODES = 10000
N_EDGES = 160000
N_PIX = 262144
D = 33
DP = 48
DG = 128
NT = 10240

NC, NS = 2, 16
NW = NC * NS
PPT = N_PIX // NW
PCH = 2048
ECH = 128
TOTCH = N_EDGES // ECH
BASECH = TOTCH // NW
EXTRACH = TOTCH - BASECH * NW
SL = NT // NS

TE_BLK = 2000
SLOPE = 0.01


@functools.cache
def _mesh():
    return plsc.VectorSubcoreMesh(
        core_axis_name="c", subcore_axis_name="s",
        num_cores=NC, num_subcores=NS)


def _lrelu(x):
    return jnp.where(x >= 0, x, SLOPE * x)


def _pix_body(val_h, idx_h, osum_h, ocnt_h,
              acc_s, acc_c, idxv, valv, tmp, res, sh_s, sh_c, sem):
    c = lax.axis_index("c")
    s = lax.axis_index("s")
    wid = s * NC + c
    zero16 = jnp.zeros((16,), F32)
    ones16 = jnp.ones((16,), F32)

    def zrow(i, cc):
        acc_s[pl.ds(i * 16, 16)] = zero16
        acc_c[pl.ds(i * 16, 16)] = zero16
        return cc
    lax.fori_loop(0, NT // 16, zrow, 0)

    def chunk(k, cc):
        b = wid * PPT + k * PCH
        pltpu.sync_copy(idx_h.at[pl.ds(b, PCH)], idxv)
        pltpu.sync_copy(val_h.at[pl.ds(b, PCH)], valv)

        def inner(i, c2):
            i16 = idxv[pl.ds(i * 16, 16)]
            v16 = valv[pl.ds(i * 16, 16)]
            plsc.addupdate_scatter(acc_s, [i16], v16)
            plsc.addupdate_scatter(acc_c, [i16], ones16)
            return c2
        lax.fori_loop(0, PCH // 16, inner, 0)
        return cc
    lax.fori_loop(0, PPT // PCH, chunk, 0)

    pltpu.sync_copy(acc_s, sh_s.at[s])
    pltpu.sync_copy(acc_c, sh_c.at[s])
    plsc.subcore_barrier()

    for sh, out_h in ((sh_s, osum_h), (sh_c, ocnt_h)):
        def z2(i, cc):
            res[pl.ds(i * 16, 16)] = zero16
            return cc
        lax.fori_loop(0, SL // 16, z2, 0)
        for j in range(NS):
            pltpu.sync_copy(sh.at[j, pl.ds(s * SL, SL)], tmp)

            def addv(i, cc):
                sl = pl.ds(i * 16, 16)
                res[sl] = res[sl] + tmp[sl]
                return cc
            lax.fori_loop(0, SL // 16, addv, 0)
        pltpu.sync_copy(res, out_h.at[c, pl.ds(s * SL, SL)])


@functools.cache
def _pix_kernel():
  return pl.kernel(
    _pix_body,
    out_type=(jax.ShapeDtypeStruct((NC, NT), F32),
              jax.ShapeDtypeStruct((NC, NT), F32)),
    mesh=_mesh(),
    scratch_types=[
        pltpu.VMEM((NT,), F32),
        pltpu.VMEM((NT,), F32),
        pltpu.VMEM((PCH,), I32),
        pltpu.VMEM((PCH,), F32),
        pltpu.VMEM((SL,), F32),
        pltpu.VMEM((SL,), F32),
        pltpu.VMEM_SHARED((NS, NT), F32),
        pltpu.VMEM_SHARED((NS, NT), F32),
        pltpu.SemaphoreType.DMA,
    ],
    compiler_params=pltpu.CompilerParams(needs_layout_passes=False),
  )


def _pix_call(val, idx):
    return _pix_kernel()(val, idx)


def _scat_body(m_h, src_h, dst_h, ang_h, out_h,
               srcv, dstv, angv, rows, zbuf, sh, sem):
    c = lax.axis_index("c")
    s = lax.axis_index("s")
    wid = s * NC + c
    zero16 = jnp.zeros((16,), F32)

    def z(i, cc):
        for kk in range(DG // 16):
            zbuf[i, pl.ds(kk * 16, 16)] = zero16
        return cc
    lax.fori_loop(0, ECH, z, 0)

    @pl.when(s == 0)
    def _():
        for j in range(NT // ECH):
            pltpu.sync_copy(zbuf, sh.at[pl.ds(j * ECH, ECH)])
    plsc.subcore_barrier()

    nch = BASECH + jnp.where(wid < EXTRACH, 1, 0)

    def chunk(k, cc):
        b = ECH * (wid + NW * k)
        pltpu.sync_copy(src_h.at[pl.ds(b, ECH)], srcv)
        pltpu.sync_copy(dst_h.at[pl.ds(b, ECH)], dstv)
        pltpu.sync_copy(ang_h.at[pl.ds(b, ECH)], angv)
        pltpu.async_copy(m_h.at[srcv], rows, sem).wait()

        def sc_g(j, c2):
            a16 = angv[pl.ds(j * 16, 16)]
            for lane in range(16):
                a = a16[lane]
                e = j * 16 + lane
                for kk in range(DG // 16):
                    sl = pl.ds(kk * 16, 16)
                    rows[e, sl] = rows[e, sl] * a
            return c2
        lax.fori_loop(0, ECH // 16, sc_g, 0)
        pltpu.sync_copy(rows, sh.at[dstv], add=True)
        return cc
    lax.fori_loop(0, nch, chunk, 0)
    plsc.subcore_barrier()

    @pl.when(s == 0)
    def _():
        pltpu.sync_copy(sh, out_h.at[c])


@functools.cache
def _scat_kernel():
  return pl.kernel(
    _scat_body,
    out_type=jax.ShapeDtypeStruct((NC, NT, DG), F32),
    mesh=_mesh(),
    scratch_types=[
        pltpu.VMEM((ECH,), I32),
        pltpu.VMEM((ECH,), I32),
        pltpu.VMEM((ECH,), F32),
        pltpu.VMEM((ECH, DG), F32),
        pltpu.VMEM((ECH, DG), F32),
        pltpu.VMEM_SHARED((NT, DG), F32),
        pltpu.SemaphoreType.DMA,
    ],
    compiler_params=pltpu.CompilerParams(needs_layout_passes=False),
  )


def _scat_call(m, srci, dsti, ang):
    return _scat_kernel()(m, srci, dsti, ang)


def _pgath_body(a_h, b_h, src_h, dst_h, ga_h, gb_h, idxv, rows, sem):
    c = lax.axis_index("c")
    s = lax.axis_index("s")
    wid = s * NC + c
    nch = BASECH + jnp.where(wid < EXTRACH, 1, 0)

    def chunk(k, cc):
        b = ECH * (wid + NW * k)
        pltpu.sync_copy(src_h.at[pl.ds(b, ECH)], idxv)
        pltpu.async_copy(a_h.at[idxv], rows, sem).wait()
        pltpu.sync_copy(rows, ga_h.at[pl.ds(b, ECH)])
        pltpu.sync_copy(dst_h.at[pl.ds(b, ECH)], idxv)
        pltpu.async_copy(b_h.at[idxv], rows, sem).wait()
        pltpu.sync_copy(rows, gb_h.at[pl.ds(b, ECH)])
        return cc
    lax.fori_loop(0, nch, chunk, 0)


@functools.cache
def _pgath_kernel():
  return pl.kernel(
    _pgath_body,
    out_type=(jax.ShapeDtypeStruct((N_EDGES, DG), F32),
              jax.ShapeDtypeStruct((N_EDGES, DG), F32)),
    mesh=_mesh(),
    scratch_types=[
        pltpu.VMEM((ECH,), I32),
        pltpu.VMEM((ECH, DG), F32),
        pltpu.SemaphoreType.DMA,
    ],
    compiler_params=pltpu.CompilerParams(needs_layout_passes=False),
  )


def _pgath_call(a, b, srci, dsti):
    return _pgath_kernel()(a, b, srci, dsti)


def _t0_body(ps0, ps1, pc0, pc1, wfe, bfe, rn, wmsg, wself, s1x_o, m1_o):
    ssum = ps0[...] + ps1[...]
    cnt = pc0[...] + pc1[...]
    x0 = (ssum * wfe[0:1, :] + cnt * bfe[0:1, :]) / jnp.maximum(cnt, 1.0)
    x0 = x0 + rn[0:1, :]
    m1_o[...] = jnp.dot(x0, wmsg[...], preferred_element_type=F32)
    s1x_o[...] = jnp.dot(x0, wself[...], preferred_element_type=F32)


def _t1_body(s1x, a1, a2, wmsg, wself, x1_o, m2_o, s2x_o):
    x1 = _lrelu(s1x[...] + (a1[...] + a2[...])[:, 0:DP])
    x1_o[...] = x1
    m2_o[...] = jnp.dot(x1, wmsg[...], preferred_element_type=F32)
    s2x_o[...] = jnp.dot(x1, wself[...], preferred_element_type=F32)


def _t3_body(s2x, a1, a2, x1, we1a, we1b, we2a, we2b, a_o, b_o):
    x2 = _lrelu(s2x[...] + (a1[...] + a2[...])[:, 0:DP])
    x1v = x1[...]
    zpad = jnp.zeros((NT, DG - 2 * DP), F32)
    a_o[...] = jnp.concatenate(
        [jnp.dot(x1v, we1a[...], preferred_element_type=F32),
         jnp.dot(x2, we2a[...], preferred_element_type=F32), zpad], axis=1)
    b_o[...] = jnp.concatenate(
        [jnp.dot(x1v, we1b[...], preferred_element_type=F32),
         jnp.dot(x2, we2b[...], preferred_element_type=F32), zpad], axis=1)


def _kw_body(w1cat, w2, w3, w4, w5p, b1, b2, b3, b4, b5, pcat_o, pb_o):
    s4 = jnp.dot(w4[...], w5p[...], preferred_element_type=F32)
    s3 = jnp.dot(w3[...], s4, preferred_element_type=F32)
    s2 = jnp.dot(w2[...], s3, preferred_element_type=F32)
    pcat_o[...] = jnp.dot(w1cat[...], s2, preferred_element_type=F32)
    pb_o[...] = (jnp.dot(b1[...], s2, preferred_element_type=F32)
                 + jnp.dot(b2[...], s3, preferred_element_type=F32)
                 + jnp.dot(b3[...], s4, preferred_element_type=F32)
                 + jnp.dot(b4[...], w5p[...], preferred_element_type=F32)
                 + b5[...])


def _t4_body(ga, gb, w8, f1d, c1p, be1p, c2p, d2p, be2p,
             pcat, pb, qcat, qb, p_o, q_o, v_o, sl_o):
    gav = ga[...]
    gbv = gb[...]
    w8v = w8[...]
    ef1 = _lrelu(gav[:, 0:DP] + gbv[:, 0:DP]
                 + jnp.dot(w8v, c1p[...], preferred_element_type=F32)
                 + be1p[0:1, :])
    sl1 = jnp.sum(ef1 * ef1)
    t = (gav[:, DP:2 * DP] + gbv[:, DP:2 * DP]
         + jnp.dot(w8v, c2p[...], preferred_element_type=F32)
         + jnp.dot(ef1, d2p[...], preferred_element_type=F32)
         + be2p[0:1, :])
    sl2 = jnp.sum(t * t)
    ef2 = _lrelu(t)
    hcat = jnp.concatenate([ef2, f1d[...], w8v], axis=1)
    lp = jnp.dot(hcat, pcat[...], preferred_element_type=F32) + pb[0:1, :]
    lq = jnp.dot(hcat, qcat[...], preferred_element_type=F32) + qb[0:1, :]
    lane = lax.broadcasted_iota(I32, (TE_BLK, 128), 1)
    msk = lane < 3
    lpm = jnp.where(msk, lp, -1e30)
    mx = jnp.max(lpm, axis=1, keepdims=True)
    ex = jnp.where(msk, jnp.exp(lp - mx), 0.0)
    sm = jnp.sum(ex, axis=1, keepdims=True)
    p = ex / sm
    v = jnp.sum(p * jnp.where(msk, lq, 0.0), axis=1, keepdims=True)
    p_o[...] = p[:, 0:8]
    q_o[...] = lq[:, 0:8]
    v_o[...] = jnp.broadcast_to(v, (TE_BLK, 8))
    r2 = lax.broadcasted_iota(I32, (8, 128), 0)
    l2 = lax.broadcasted_iota(I32, (8, 128), 1)
    upd = (jnp.where((r2 == 0) & (l2 == 0), sl1, 0.0)
           + jnp.where((r2 == 0) & (l2 == 1), sl2, 0.0))

    @pl.when(pl.program_id(0) == 0)
    def _():
        sl_o[...] = jnp.zeros((8, 128), F32)
    sl_o[...] += upd


def _t0_call(ps0, ps1, pc0, pc1, wfe, bfe, rn, wmsg, wself):
    return pl.pallas_call(
        _t0_body,
        out_shape=(jax.ShapeDtypeStruct((NT, DP), F32),
                   jax.ShapeDtypeStruct((NT, DG), F32)),
    )(ps0, ps1, pc0, pc1, wfe, bfe, rn, wmsg, wself)


def _t1_call(s1x, a1, a2, wmsg, wself):
    return pl.pallas_call(
        _t1_body,
        out_shape=(jax.ShapeDtypeStruct((NT, DP), F32),
                   jax.ShapeDtypeStruct((NT, DG), F32),
                   jax.ShapeDtypeStruct((NT, DP), F32)),
    )(s1x, a1, a2, wmsg, wself)


def _t3_call(s2x, a1, a2, x1, we1a, we1b, we2a, we2b):
    return pl.pallas_call(
        _t3_body,
        out_shape=(jax.ShapeDtypeStruct((NT, DG), F32),
                   jax.ShapeDtypeStruct((NT, DG), F32)),
    )(s2x, a1, a2, x1, we1a, we1b, we2a, we2b)


def _kw_call(w1cat, w2, w3, w4, w5p, b1, b2, b3, b4, b5):
    return pl.pallas_call(
        _kw_body,
        out_shape=(jax.ShapeDtypeStruct((DP + 16 + 8, 128), F32),
                   jax.ShapeDtypeStruct((8, 128), F32)),
    )(w1cat, w2, w3, w4, w5p, b1, b2, b3, b4, b5)


def _t4_call(ga, gb, w8, f1d, c1p, be1p, c2p, d2p, be2p, pcat, pb, qcat, qb):
    ng = N_EDGES // TE_BLK
    eb = lambda cols: pl.BlockSpec((TE_BLK, cols), lambda i: (i, 0))
    wb = lambda r, c: pl.BlockSpec((r, c), lambda i: (0, 0))
    return pl.pallas_call(
        _t4_body,
        grid=(ng,),
        in_specs=[eb(DG), eb(DG), eb(8), eb(16),
                  wb(8, DP), wb(8, DP), wb(8, DP), wb(DP, DP), wb(8, DP),
                  wb(DP + 16 + 8, 128), wb(8, 128),
                  wb(DP + 16 + 8, 128), wb(8, 128)],
        out_specs=[eb(8), eb(8), eb(8),
                   pl.BlockSpec((8, 128), lambda i: (0, 0))],
        out_shape=(jax.ShapeDtypeStruct((N_EDGES, 8), F32),
                   jax.ShapeDtypeStruct((N_EDGES, 8), F32),
                   jax.ShapeDtypeStruct((N_EDGES, 8), F32),
                   jax.ShapeDtypeStruct((8, 128), F32)),
    )(ga, gb, w8, f1d, c1p, be1p, c2p, d2p, be2p, pcat, pb, qcat, qb)


def _pad2(a, r, c):
    return jnp.pad(a, ((0, r - a.shape[0]), (0, c - a.shape[1])))


def _row8(v, c):
    return jnp.pad(v[None, :], ((0, 7), (0, c - v.shape[0])))


def kernel(image, edge_weights, angles, edge_features_1d, W_fe, b_fe,
           Wself1, Wmsg1, We1, be1, Wself2, Wmsg2, We2, be2,
           p_params, q_params, sp_indices, edge_index, round_n):
    img_flat = image.reshape(-1).astype(F32)
    spi = jnp.asarray(sp_indices, I32)
    ei = jnp.asarray(edge_index, I32)
    srci, dsti = ei[0], ei[1]
    ang = jnp.asarray(angles, F32)
    w_e = jnp.asarray(edge_weights, F32)
    w8 = jnp.pad(w_e[:, None], ((0, 0), (0, 7)))
    f1d = jnp.asarray(edge_features_1d, F32)

    wfe_row = _row8(W_fe[0], DP)
    bfe_row = _row8(b_fe, DP)
    rn = jnp.asarray(round_n, F32)
    rn_row = jnp.zeros((8, DP), F32).at[0, D - 1].set(rn)

    wself1p = _pad2(Wself1, DP, DP)
    wmsg1p = _pad2(Wmsg1, DP, DG)
    wself2p = _pad2(Wself2, DP, DP)
    wmsg2p = _pad2(Wmsg2, DP, DG)

    we1a = _pad2(We1[0:D], DP, DP)
    we1b = _pad2(We1[D:2 * D], DP, DP)
    c1p = _pad2(We1[2 * D:2 * D + 1], 8, DP)
    be1p = _row8(be1, DP)
    we2a = _pad2(We2[0:D], DP, DP)
    we2b = _pad2(We2[D:2 * D], DP, DP)
    c2p = _pad2(We2[2 * D:2 * D + 1], 8, DP)
    d2p = _pad2(We2[2 * D + 1:3 * D + 1], DP, DP)
    be2p = _row8(be2, DP)

    def head_weights(params):
        (w1, b1), (w2, b2), (w3, b3), (w4, b4), (w5, b5) = params
        w1cat_pre = jnp.concatenate(
            [_pad2(w1[0:D], DP, 256), w1[D:D + 16], _pad2(w1[D + 16:], 8, 256)],
            axis=0)
        return _kw_call(w1cat_pre, w2, w3, w4, _pad2(w5, 256, 128),
                        _row8(b1, 256), _row8(b2, 512), _row8(b3, 1024),
                        _row8(b4, 256), _row8(b5, 128))

    pcat, pb = head_weights(p_params)
    qcat, qb = head_weights(q_params)

    ps, pc = _pix_call(img_flat, spi)
    s1x, m1 = _t0_call(ps[0].reshape(NT, 1), ps[1].reshape(NT, 1),
                       pc[0].reshape(NT, 1), pc[1].reshape(NT, 1),
                       wfe_row, bfe_row, rn_row, wmsg1p, wself1p)

    agg1 = _scat_call(m1, srci, dsti, ang)
    x1, m2, s2x = _t1_call(s1x, agg1[0], agg1[1], wmsg2p, wself2p)

    agg2 = _scat_call(m2, srci, dsti, ang)
    a_tab, b_tab = _t3_call(s2x, agg2[0], agg2[1], x1,
                            we1a, we1b, we2a, we2b)
    ga, gb = _pgath_call(a_tab, b_tab, srci, dsti)

    p8, q8, v8, sl = _t4_call(ga, gb, w8, f1d, c1p, be1p, c2p, d2p, be2p,
                              pcat, pb, qcat, qb)
    p = p8[:, 0:3]
    q = q8[:, 0:3]
    v = v8[:, 0]
    side_loss = 0.5 * (sl[0, 0] + sl[0, 1]) / (N_EDGES * D)
    return (p, q, v, side_loss)
